# Initial kernel scaffold; baseline (speedup 1.0000x reference)
import jax
import jax.numpy as jnp
from jax.experimental import pallas as pl


def kernel(x, edge_index, batch, W1, b1, bn1_gamma, bn1_beta, bn1_mean, bn1_var, W2, b2, lin_W, lin_b):
    raise NotImplementedError("write your pallas kernel here")


# speedup vs baseline: 45.6908x; 45.6908x over previous
import functools

import jax
import jax.numpy as jnp
from jax import lax
from jax.experimental import pallas as pl
from jax.experimental.pallas import tpu as pltpu
from jax.experimental.pallas import tpu_sc as plsc

N = 10000
E = 320000
HID = 256
NG = 128
OUT_DIM = 128
EPS = 1e-5

NC = 2
NS = 16
NW = NC * NS
L = 16

EW = E // NW
CH = 400
VPC = CH // L
NCHUNK = EW // CH
CCH = 80
CVPC = CCH // L
CNCHUNK = EW // CCH
CSL = NG * N // NS

BN = 2048
NP = 10240
NSTEP = NP // BN


def _sc_degrees_body(src_hbm, dst_hbm, zeros_hbm, dps_hbm, dpd_hbm,
                     src_v, dst_v, dso_v, dsi_v):
    cid = lax.axis_index("c")
    sid = lax.axis_index("s")
    wid = sid * NC + cid
    base = wid * EW
    pltpu.sync_copy(zeros_hbm, dso_v)
    pltpu.sync_copy(zeros_hbm, dsi_v)
    ones = jnp.full((L,), 1.0, jnp.float32)

    def chunk(k, _):
        off = base + k * CH
        pltpu.sync_copy(src_hbm.at[pl.ds(off, CH)], src_v)
        pltpu.sync_copy(dst_hbm.at[pl.ds(off, CH)], dst_v)
        for j in range(VPC):
            s16 = src_v[pl.ds(j * L, L)]
            d16 = dst_v[pl.ds(j * L, L)]
            plsc.addupdate_scatter(dso_v, [s16], ones)
            plsc.addupdate_scatter(dsi_v, [d16], ones)
        return 0

    lax.fori_loop(0, NCHUNK, chunk, 0)
    pltpu.sync_copy(dso_v, dps_hbm.at[wid])
    pltpu.sync_copy(dsi_v, dpd_hbm.at[wid])


def _sc_degrees(src, dst, zeros_n):
    mesh = plsc.VectorSubcoreMesh(core_axis_name="c", subcore_axis_name="s")
    f = pl.kernel(
        _sc_degrees_body,
        out_type=[jax.ShapeDtypeStruct((NW, N), jnp.float32),
                  jax.ShapeDtypeStruct((NW, N), jnp.float32)],
        mesh=mesh,
        compiler_params=pltpu.CompilerParams(needs_layout_passes=False, use_tc_tiling_on_sc=False),
        scratch_types=[
            pltpu.VMEM((CH,), jnp.int32),
            pltpu.VMEM((CH,), jnp.int32),
            pltpu.VMEM((N,), jnp.float32),
            pltpu.VMEM((N,), jnp.float32),
        ],
    )
    return f(src, dst, zeros_n)


def _tc_prep_body(dps_ref, dpd_ref, batch_ref, dinv_ref, dego_ref, cnt_ref):
    dso = jnp.sum(dps_ref[...], axis=0, keepdims=True)
    dsi = jnp.sum(dpd_ref[...], axis=0, keepdims=True) + 1.0
    dego_ref[...] = dso
    dinv_ref[...] = lax.rsqrt(dsi)
    bt = batch_ref[...]
    gi = lax.broadcasted_iota(jnp.int32, (NG, N), 0)
    m = (gi == bt).astype(jnp.float32)
    cnt_ref[...] = jnp.sum(m, axis=1, keepdims=True)


def _tc_prep(dps, dpd, batch2d):
    return pl.pallas_call(
        _tc_prep_body,
        out_shape=[jax.ShapeDtypeStruct((1, N), jnp.float32),
                   jax.ShapeDtypeStruct((1, N), jnp.float32),
                   jax.ShapeDtypeStruct((NG, 1), jnp.float32)],
    )(dps, dpd, batch2d)


def _sc_edges_body(src_hbm, dst_hbm, dinv_hbm, dego_hbm, batch_hbm, zeros_hbm,
                   tpart_hbm, cpart_hbm,
                   src_v, dst_v, dinv_v, dego_v, batch_v, t_v, cval_v, cidx_v,
                   cshared):
    cid = lax.axis_index("c")
    sid = lax.axis_index("s")
    wid = sid * NC + cid
    base = wid * EW
    pltpu.sync_copy(dinv_hbm, dinv_v)
    pltpu.sync_copy(dego_hbm, dego_v)
    pltpu.sync_copy(batch_hbm, batch_v)
    pltpu.sync_copy(zeros_hbm, t_v)
    for r in range(CSL // N):
        pltpu.sync_copy(zeros_hbm, cshared.at[pl.ds(sid * CSL + r * N, N)])
    plsc.subcore_barrier()

    def chunk(k, _):
        off = base + k * CCH
        pltpu.sync_copy(src_hbm.at[pl.ds(off, CCH)], src_v)
        pltpu.sync_copy(dst_hbm.at[pl.ds(off, CCH)], dst_v)
        for j in range(CVPC):
            s16 = src_v[pl.ds(j * L, L)]
            d16 = dst_v[pl.ds(j * L, L)]
            dis = plsc.load_gather(dinv_v, [s16])
            did = plsc.load_gather(dinv_v, [d16])
            go = plsc.load_gather(dego_v, [s16])
            bt = plsc.load_gather(batch_v, [d16])
            nrm = dis * did
            plsc.addupdate_scatter(t_v, [d16], nrm * go)
            cidx_v[pl.ds(j * L, L)] = bt * N + s16
            cval_v[pl.ds(j * L, L)] = nrm
        pltpu.sync_copy(cval_v, cshared.at[cidx_v], add=True)
        return 0

    lax.fori_loop(0, CNCHUNK, chunk, 0)
    pltpu.sync_copy(t_v, tpart_hbm.at[wid])
    plsc.subcore_barrier()
    pltpu.sync_copy(cshared.at[pl.ds(sid * CSL, CSL)],
                    cpart_hbm.at[cid, pl.ds(sid * CSL, CSL)])


def _sc_edges(src, dst, dinv, dego, batch, zeros_n):
    mesh = plsc.VectorSubcoreMesh(core_axis_name="c", subcore_axis_name="s")
    f = pl.kernel(
        _sc_edges_body,
        out_type=[jax.ShapeDtypeStruct((NW, N), jnp.float32),
                  jax.ShapeDtypeStruct((NC, NG * N), jnp.float32)],
        mesh=mesh,
        compiler_params=pltpu.CompilerParams(needs_layout_passes=False, use_tc_tiling_on_sc=False),
        scratch_types=[
            pltpu.VMEM((CCH,), jnp.int32),
            pltpu.VMEM((CCH,), jnp.int32),
            pltpu.VMEM((N,), jnp.float32),
            pltpu.VMEM((N,), jnp.float32),
            pltpu.VMEM((N,), jnp.int32),
            pltpu.VMEM((N,), jnp.float32),
            pltpu.VMEM((CCH,), jnp.float32),
            pltpu.VMEM((CCH,), jnp.int32),
            pltpu.VMEM_SHARED((NG * N,), jnp.float32),
        ],
    )
    return f(src, dst, dinv, dego, batch, zeros_n)


def _tc_final_body(tpart_ref, cpart_ref, dinv_ref, dego_ref, batch_ref,
                   g1_ref, g0_ref, W2_ref, b2_ref, linW_ref, linb_ref,
                   cnt_ref, out_ref, q_acc):
    i = pl.program_id(0)

    @pl.when(i == 0)
    def _():
        q_acc[...] = jnp.zeros_like(q_acc)

    dinv = dinv_ref[...]
    dinv2 = dinv * dinv
    t = jnp.sum(tpart_ref[...], axis=0, keepdims=True) + dinv2 * dego_ref[...]
    hT = jnp.maximum(g1_ref[...] * t + g0_ref[...], 0.0)
    gi = lax.broadcasted_iota(jnp.int32, (NG, BN), 0)
    m = (gi == batch_ref[...]).astype(jnp.float32)
    c_tot = cpart_ref[0] + cpart_ref[1] + m * dinv2
    q_acc[...] += lax.dot_general(
        c_tot, hT, (((1,), (1,)), ((), ())),
        preferred_element_type=jnp.float32)

    @pl.when(i == NSTEP - 1)
    def _():
        cnt = cnt_ref[...]
        maxcnt = jnp.maximum(cnt, 1.0)
        pooled = (jnp.dot(q_acc[...], W2_ref[...],
                          preferred_element_type=jnp.float32)
                  + cnt * b2_ref[...]) / maxcnt
        out_ref[...] = (jnp.dot(pooled, linW_ref[...],
                                preferred_element_type=jnp.float32)
                        + linb_ref[...])


def _tc_final(tpart, cpart, dinv, dego, batch2d, g1c, g0c, W2, b2r, lin_W,
              lin_br, cnt):
    grid = (NSTEP,)
    return pl.pallas_call(
        _tc_final_body,
        grid=grid,
        in_specs=[
            pl.BlockSpec((NW, BN), lambda i: (0, i)),
            pl.BlockSpec((NC, NG, BN), lambda i: (0, 0, i)),
            pl.BlockSpec((1, BN), lambda i: (0, i)),
            pl.BlockSpec((1, BN), lambda i: (0, i)),
            pl.BlockSpec((1, BN), lambda i: (0, i)),
            pl.BlockSpec((HID, 1), lambda i: (0, 0)),
            pl.BlockSpec((HID, 1), lambda i: (0, 0)),
            pl.BlockSpec((HID, HID), lambda i: (0, 0)),
            pl.BlockSpec((1, HID), lambda i: (0, 0)),
            pl.BlockSpec((HID, OUT_DIM), lambda i: (0, 0)),
            pl.BlockSpec((1, OUT_DIM), lambda i: (0, 0)),
            pl.BlockSpec((NG, 1), lambda i: (0, 0)),
        ],
        out_specs=pl.BlockSpec((NG, OUT_DIM), lambda i: (0, 0)),
        out_shape=jax.ShapeDtypeStruct((NG, OUT_DIM), jnp.float32),
        scratch_shapes=[pltpu.VMEM((NG, HID), jnp.float32)],
    )(tpart, cpart, dinv, dego, batch2d, g1c, g0c, W2, b2r, lin_W, lin_br,
      cnt)


def kernel(x, edge_index, batch, W1, b1, bn1_gamma, bn1_beta, bn1_mean,
           bn1_var, W2, b2, lin_W, lin_b):
    src = edge_index[0]
    dst = edge_index[1]
    zeros_n = jnp.zeros((N,), jnp.float32)

    dps, dpd = _sc_degrees(src, dst, zeros_n)
    dinv, dego, cnt = _tc_prep(dps, dpd, batch.reshape(1, N))
    tpart, cpart = _sc_edges(src, dst, dinv.reshape(N), dego.reshape(N),
                             batch, zeros_n)

    bscale = bn1_gamma * lax.rsqrt(bn1_var + EPS)
    g1c = (W1[0] * bscale).reshape(HID, 1)
    g0c = ((b1 - bn1_mean) * bscale + bn1_beta).reshape(HID, 1)

    pad = NP - N
    tpart_p = jnp.pad(tpart, ((0, 0), (0, pad)))
    cpart_p = jnp.pad(cpart.reshape(NC, NG, N), ((0, 0), (0, 0), (0, pad)))
    dinv_p = jnp.pad(dinv, ((0, 0), (0, pad)))
    dego_p = jnp.pad(dego, ((0, 0), (0, pad)))
    batch_p = jnp.pad(batch.reshape(1, N), ((0, 0), (0, pad)),
                      constant_values=-1)

    return _tc_final(tpart_p, cpart_p, dinv_p, dego_p, batch_p, g1c, g0c,
                     W2, b2.reshape(1, HID), lin_W, lin_b.reshape(1, OUT_DIM),
                     cnt)


# speedup vs baseline: 89.7123x; 1.9635x over previous
import jax
import jax.numpy as jnp
from jax import lax
from jax.experimental import pallas as pl
from jax.experimental.pallas import tpu as pltpu
from jax.experimental.pallas import tpu_sc as plsc

N = 10000
E = 320000
HID = 256
NG = 128
OUT_DIM = 128
EPS = 1e-5

NC = 2
NS = 16
NW = NC * NS
L = 16
RING = 5

EW = E // NW
ACH = 400
AVPC = ACH // L
ANK = EW // (ACH * RING)
CCH = 80
CVPC = CCH // L
CNK = EW // (CCH * RING)
CSL = NG * N // NS

BN = 2048
NP = 10240
NSTEP = NP // BN

_SC_PARAMS = pltpu.CompilerParams(needs_layout_passes=False,
                                  use_tc_tiling_on_sc=False)


def _sc_degrees_body(src_hbm, dst_hbm, zeros_hbm, dps_hbm, dpd_hbm,
                     sb0, sb1, sb2, sb3, sb4, db0, db1, db2, db3, db4,
                     dso_v, dsi_v, sm0, sm1, sm2, sm3, sm4):
    sbufs = (sb0, sb1, sb2, sb3, sb4)
    dbufs = (db0, db1, db2, db3, db4)
    sems = (sm0, sm1, sm2, sm3, sm4)
    cid = lax.axis_index("c")
    sid = lax.axis_index("s")
    wid = sid * NC + cid
    base = wid * EW
    pltpu.sync_copy(zeros_hbm, dso_v)
    pltpu.sync_copy(zeros_hbm, dsi_v)
    ones = jnp.full((L,), 1.0, jnp.float32)
    for b in range(RING):
        off = base + b * ACH
        pltpu.async_copy(src_hbm.at[pl.ds(off, ACH)], sbufs[b], sems[b])
        pltpu.async_copy(dst_hbm.at[pl.ds(off, ACH)], dbufs[b], sems[b])

    def round_(k, _):
        for b in range(RING):
            off = base + (k * RING + b) * ACH
            pltpu.make_async_copy(src_hbm.at[pl.ds(off, ACH)], sbufs[b],
                                  sems[b]).wait()
            pltpu.make_async_copy(dst_hbm.at[pl.ds(off, ACH)], dbufs[b],
                                  sems[b]).wait()
            for j in range(AVPC):
                s16 = sbufs[b][pl.ds(j * L, L)]
                d16 = dbufs[b][pl.ds(j * L, L)]
                plsc.addupdate_scatter(dso_v, [s16], ones)
                plsc.addupdate_scatter(dsi_v, [d16], ones)

            @pl.when(k < ANK - 1)
            def _():
                noff = base + ((k + 1) * RING + b) * ACH
                pltpu.async_copy(src_hbm.at[pl.ds(noff, ACH)], sbufs[b],
                                 sems[b])
                pltpu.async_copy(dst_hbm.at[pl.ds(noff, ACH)], dbufs[b],
                                 sems[b])
        return 0

    lax.fori_loop(0, ANK, round_, 0)
    pltpu.sync_copy(dso_v, dps_hbm.at[wid])
    pltpu.sync_copy(dsi_v, dpd_hbm.at[wid])


def _sc_degrees(src, dst, zeros_n):
    mesh = plsc.VectorSubcoreMesh(core_axis_name="c", subcore_axis_name="s")
    f = pl.kernel(
        _sc_degrees_body,
        out_type=[jax.ShapeDtypeStruct((NW, N), jnp.float32),
                  jax.ShapeDtypeStruct((NW, N), jnp.float32)],
        mesh=mesh,
        compiler_params=_SC_PARAMS,
        scratch_types=(
            [pltpu.VMEM((ACH,), jnp.int32) for _ in range(2 * RING)]
            + [pltpu.VMEM((N,), jnp.float32) for _ in range(2)]
            + [pltpu.SemaphoreType.DMA for _ in range(RING)]
        ),
    )
    return f(src, dst, zeros_n)


def _tc_prep_body(dps_ref, dpd_ref, batch_ref, dinv_ref, dego_ref, w_ref,
                  bN_ref, cnt_ref):
    dso = jnp.sum(dps_ref[...], axis=0, keepdims=True)
    dsi = jnp.sum(dpd_ref[...], axis=0, keepdims=True) + 1.0
    dinv = lax.rsqrt(dsi)
    dego_ref[...] = dso
    dinv_ref[...] = dinv
    w_ref[...] = dinv * dso
    bt = batch_ref[...]
    bN_ref[...] = bt * N
    gi = lax.broadcasted_iota(jnp.int32, (NG, N), 0)
    m = (gi == bt).astype(jnp.float32)
    cnt_ref[...] = jnp.sum(m, axis=1, keepdims=True)


def _tc_prep(dps, dpd, batch2d):
    return pl.pallas_call(
        _tc_prep_body,
        out_shape=[jax.ShapeDtypeStruct((1, N), jnp.float32),
                   jax.ShapeDtypeStruct((1, N), jnp.float32),
                   jax.ShapeDtypeStruct((1, N), jnp.float32),
                   jax.ShapeDtypeStruct((1, N), jnp.int32),
                   jax.ShapeDtypeStruct((NG, 1), jnp.float32)],
    )(dps, dpd, batch2d)


def _sc_edges_body(src_hbm, dst_hbm, dinv_hbm, w_hbm, bN_hbm, zeros_hbm,
                   upart_hbm, mpart_hbm,
                   sb0, sb1, sb2, sb3, sb4, db0, db1, db2, db3, db4,
                   ci0, ci1, ci2, ci3, ci4, cv0, cv1, cv2, cv3, cv4,
                   dinv_v, w_v, bN_v, u_v, cshared,
                   sm0, sm1, sm2, sm3, sm4, cm0, cm1, cm2, cm3, cm4):
    sbufs = (sb0, sb1, sb2, sb3, sb4)
    dbufs = (db0, db1, db2, db3, db4)
    cidx = (ci0, ci1, ci2, ci3, ci4)
    cval = (cv0, cv1, cv2, cv3, cv4)
    sems = (sm0, sm1, sm2, sm3, sm4)
    csems = (cm0, cm1, cm2, cm3, cm4)
    cid = lax.axis_index("c")
    sid = lax.axis_index("s")
    wid = sid * NC + cid
    base = wid * EW
    pltpu.sync_copy(dinv_hbm, dinv_v)
    pltpu.sync_copy(w_hbm, w_v)
    pltpu.sync_copy(bN_hbm, bN_v)
    pltpu.sync_copy(zeros_hbm, u_v)
    for r in range(CSL // N):
        pltpu.sync_copy(zeros_hbm, cshared.at[pl.ds(sid * CSL + r * N, N)])
    for b in range(RING):
        off = base + b * CCH
        pltpu.async_copy(src_hbm.at[pl.ds(off, CCH)], sbufs[b], sems[b])
        pltpu.async_copy(dst_hbm.at[pl.ds(off, CCH)], dbufs[b], sems[b])
    plsc.subcore_barrier()

    def round_(k, _):
        for b in range(RING):
            off = base + (k * RING + b) * CCH
            pltpu.make_async_copy(src_hbm.at[pl.ds(off, CCH)], sbufs[b],
                                  sems[b]).wait()
            pltpu.make_async_copy(dst_hbm.at[pl.ds(off, CCH)], dbufs[b],
                                  sems[b]).wait()

            @pl.when(k > 0)
            def _():
                pltpu.make_async_copy(cval[b], cshared.at[cidx[b]],
                                      csems[b]).wait()

            for j in range(CVPC):
                s16 = sbufs[b][pl.ds(j * L, L)]
                d16 = dbufs[b][pl.ds(j * L, L)]
                ws = plsc.load_gather(w_v, [s16])
                dd = plsc.load_gather(dinv_v, [d16])
                bd = plsc.load_gather(bN_v, [d16])
                plsc.addupdate_scatter(u_v, [d16], ws)
                cidx[b][pl.ds(j * L, L)] = bd + s16
                cval[b][pl.ds(j * L, L)] = dd
            pltpu.async_copy(cval[b], cshared.at[cidx[b]], csems[b],
                             add=True)

            @pl.when(k < CNK - 1)
            def _():
                noff = base + ((k + 1) * RING + b) * CCH
                pltpu.async_copy(src_hbm.at[pl.ds(noff, CCH)], sbufs[b],
                                 sems[b])
                pltpu.async_copy(dst_hbm.at[pl.ds(noff, CCH)], dbufs[b],
                                 sems[b])
        return 0

    lax.fori_loop(0, CNK, round_, 0)
    for b in range(RING):
        pltpu.make_async_copy(cval[b], cshared.at[cidx[b]], csems[b]).wait()
    pltpu.sync_copy(u_v, upart_hbm.at[wid])
    plsc.subcore_barrier()
    pltpu.sync_copy(cshared.at[pl.ds(sid * CSL, CSL)],
                    mpart_hbm.at[cid, pl.ds(sid * CSL, CSL)])


def _sc_edges(src, dst, dinv, w, bN, zeros_n):
    mesh = plsc.VectorSubcoreMesh(core_axis_name="c", subcore_axis_name="s")
    f = pl.kernel(
        _sc_edges_body,
        out_type=[jax.ShapeDtypeStruct((NW, N), jnp.float32),
                  jax.ShapeDtypeStruct((NC, NG * N), jnp.float32)],
        mesh=mesh,
        compiler_params=_SC_PARAMS,
        scratch_types=(
            [pltpu.VMEM((CCH,), jnp.int32) for _ in range(2 * RING)]
            + [pltpu.VMEM((CCH,), jnp.int32) for _ in range(RING)]
            + [pltpu.VMEM((CCH,), jnp.float32) for _ in range(RING)]
            + [pltpu.VMEM((N,), jnp.float32),
               pltpu.VMEM((N,), jnp.float32),
               pltpu.VMEM((N,), jnp.int32),
               pltpu.VMEM((N,), jnp.float32),
               pltpu.VMEM_SHARED((NG * N,), jnp.float32)]
            + [pltpu.SemaphoreType.DMA for _ in range(2 * RING)]
        ),
    )
    return f(src, dst, dinv, w, bN, zeros_n)


def _tc_final_body(upart_ref, mpart_ref, dinv_ref, dego_ref, batch_ref,
                   g1_ref, g0_ref, W2_ref, b2_ref, linW_ref, linb_ref,
                   cnt_ref, out_ref, q_acc):
    i = pl.program_id(0)

    @pl.when(i == 0)
    def _():
        q_acc[...] = jnp.zeros_like(q_acc)

    dinv = dinv_ref[...]
    dinv2 = dinv * dinv
    u = jnp.sum(upart_ref[...], axis=0, keepdims=True)
    t = dinv * u + dinv2 * dego_ref[...]
    hT = jnp.maximum(g1_ref[...] * t + g0_ref[...], 0.0)
    gi = lax.broadcasted_iota(jnp.int32, (NG, BN), 0)
    m = (gi == batch_ref[...]).astype(jnp.float32)
    c_tot = (mpart_ref[0] + mpart_ref[1]) * dinv + m * dinv2
    q_acc[...] += lax.dot_general(
        c_tot, hT, (((1,), (1,)), ((), ())),
        preferred_element_type=jnp.float32)

    @pl.when(i == NSTEP - 1)
    def _():
        cnt = cnt_ref[...]
        maxcnt = jnp.maximum(cnt, 1.0)
        pooled = (jnp.dot(q_acc[...], W2_ref[...],
                          preferred_element_type=jnp.float32)
                  + cnt * b2_ref[...]) / maxcnt
        out_ref[...] = (jnp.dot(pooled, linW_ref[...],
                                preferred_element_type=jnp.float32)
                        + linb_ref[...])


def _tc_final(upart, mpart, dinv, dego, batch2d, g1c, g0c, W2, b2r, lin_W,
              lin_br, cnt):
    return pl.pallas_call(
        _tc_final_body,
        grid=(NSTEP,),
        in_specs=[
            pl.BlockSpec((NW, BN), lambda i: (0, i)),
            pl.BlockSpec((NC, NG, BN), lambda i: (0, 0, i)),
            pl.BlockSpec((1, BN), lambda i: (0, i)),
            pl.BlockSpec((1, BN), lambda i: (0, i)),
            pl.BlockSpec((1, BN), lambda i: (0, i)),
            pl.BlockSpec((HID, 1), lambda i: (0, 0)),
            pl.BlockSpec((HID, 1), lambda i: (0, 0)),
            pl.BlockSpec((HID, HID), lambda i: (0, 0)),
            pl.BlockSpec((1, HID), lambda i: (0, 0)),
            pl.BlockSpec((HID, OUT_DIM), lambda i: (0, 0)),
            pl.BlockSpec((1, OUT_DIM), lambda i: (0, 0)),
            pl.BlockSpec((NG, 1), lambda i: (0, 0)),
        ],
        out_specs=pl.BlockSpec((NG, OUT_DIM), lambda i: (0, 0)),
        out_shape=jax.ShapeDtypeStruct((NG, OUT_DIM), jnp.float32),
        scratch_shapes=[pltpu.VMEM((NG, HID), jnp.float32)],
    )(upart, mpart, dinv, dego, batch2d, g1c, g0c, W2, b2r, lin_W, lin_br,
      cnt)


def kernel(x, edge_index, batch, W1, b1, bn1_gamma, bn1_beta, bn1_mean,
           bn1_var, W2, b2, lin_W, lin_b):
    src = edge_index[0]
    dst = edge_index[1]
    zeros_n = jnp.zeros((N,), jnp.float32)

    dps, dpd = _sc_degrees(src, dst, zeros_n)
    dinv, dego, w, bN, cnt = _tc_prep(dps, dpd, batch.reshape(1, N))
    upart, mpart = _sc_edges(src, dst, dinv.reshape(N), w.reshape(N),
                             bN.reshape(N), zeros_n)

    bscale = bn1_gamma * lax.rsqrt(bn1_var + EPS)
    g1c = (W1[0] * bscale).reshape(HID, 1)
    g0c = ((b1 - bn1_mean) * bscale + bn1_beta).reshape(HID, 1)

    pad = NP - N
    upart_p = jnp.pad(upart, ((0, 0), (0, pad)))
    mpart_p = jnp.pad(mpart.reshape(NC, NG, N), ((0, 0), (0, 0), (0, pad)))
    dinv_p = jnp.pad(dinv, ((0, 0), (0, pad)))
    dego_p = jnp.pad(dego, ((0, 0), (0, pad)))
    batch_p = jnp.pad(batch.reshape(1, N), ((0, 0), (0, pad)),
                      constant_values=-1)

    return _tc_final(upart_p, mpart_p, dinv_p, dego_p, batch_p, g1c, g0c,
                     W2, b2.reshape(1, HID), lin_W, lin_b.reshape(1, OUT_DIM),
                     cnt)


# speedup vs baseline: 98.2382x; 1.0950x over previous
import jax
import jax.numpy as jnp
from jax import lax
from jax.experimental import pallas as pl
from jax.experimental.pallas import tpu as pltpu
from jax.experimental.pallas import tpu_sc as plsc

N = 10000
E = 320000
HID = 256
NG = 128
OUT_DIM = 128
EPS = 1e-5

NC = 2
NS = 16
NW = NC * NS
L = 16
RING = 5

EW = E // NW
ACH = 400
AVPC = ACH // L
ANK = EW // (ACH * RING)
CCH = 80
CVPC = CCH // L
CNK = EW // (CCH * RING)
CSL = NG * N // NS

BN = 2048
NP = 10240
NSTEP = NP // BN

_SC_PARAMS = pltpu.CompilerParams(needs_layout_passes=False,
                                  use_tc_tiling_on_sc=False)


def _sc_degrees_body(src_hbm, dst_hbm, zeros_hbm, dps_hbm, dpd_hbm,
                     sb0, sb1, sb2, sb3, sb4, db0, db1, db2, db3, db4,
                     dso_v, dsi_v, sm0, sm1, sm2, sm3, sm4):
    sbufs = (sb0, sb1, sb2, sb3, sb4)
    dbufs = (db0, db1, db2, db3, db4)
    sems = (sm0, sm1, sm2, sm3, sm4)
    cid = lax.axis_index("c")
    sid = lax.axis_index("s")
    wid = sid * NC + cid
    base = wid * EW
    pltpu.sync_copy(zeros_hbm, dso_v)
    pltpu.sync_copy(zeros_hbm, dsi_v)
    ones = jnp.full((L,), 1.0, jnp.float32)
    for b in range(RING):
        off = base + b * ACH
        pltpu.async_copy(src_hbm.at[pl.ds(off, ACH)], sbufs[b], sems[b])
        pltpu.async_copy(dst_hbm.at[pl.ds(off, ACH)], dbufs[b], sems[b])

    def round_(k, _):
        for b in range(RING):
            off = base + (k * RING + b) * ACH
            pltpu.make_async_copy(src_hbm.at[pl.ds(off, ACH)], sbufs[b],
                                  sems[b]).wait()
            pltpu.make_async_copy(dst_hbm.at[pl.ds(off, ACH)], dbufs[b],
                                  sems[b]).wait()
            for j in range(AVPC):
                s16 = sbufs[b][pl.ds(j * L, L)]
                d16 = dbufs[b][pl.ds(j * L, L)]
                plsc.addupdate_scatter(dso_v, [s16], ones)
                plsc.addupdate_scatter(dsi_v, [d16], ones)

            @pl.when(k < ANK - 1)
            def _():
                noff = base + ((k + 1) * RING + b) * ACH
                pltpu.async_copy(src_hbm.at[pl.ds(noff, ACH)], sbufs[b],
                                 sems[b])
                pltpu.async_copy(dst_hbm.at[pl.ds(noff, ACH)], dbufs[b],
                                 sems[b])
        return 0

    lax.fori_loop(0, ANK, round_, 0)
    pltpu.sync_copy(dso_v, dps_hbm.at[wid])
    pltpu.sync_copy(dsi_v, dpd_hbm.at[wid])


def _sc_degrees(src, dst, zeros_n):
    mesh = plsc.VectorSubcoreMesh(core_axis_name="c", subcore_axis_name="s")
    f = pl.kernel(
        _sc_degrees_body,
        out_type=[jax.ShapeDtypeStruct((NW, N), jnp.float32),
                  jax.ShapeDtypeStruct((NW, N), jnp.float32)],
        mesh=mesh,
        compiler_params=_SC_PARAMS,
        scratch_types=(
            [pltpu.VMEM((ACH,), jnp.int32) for _ in range(2 * RING)]
            + [pltpu.VMEM((N,), jnp.float32) for _ in range(2)]
            + [pltpu.SemaphoreType.DMA for _ in range(RING)]
        ),
    )
    return f(src, dst, zeros_n)


def _tc_prep_body(dps_ref, dpd_ref, batch_ref, dinv_ref, dego_ref, w_ref,
                  bN_ref, cnt_ref):
    dso = jnp.sum(dps_ref[...], axis=0, keepdims=True)
    dsi = jnp.sum(dpd_ref[...], axis=0, keepdims=True) + 1.0
    dinv = lax.rsqrt(dsi)
    dego_ref[...] = dso
    dinv_ref[...] = dinv
    w_ref[...] = dinv * dso
    bt = batch_ref[...]
    bN_ref[...] = bt * N
    gi = lax.broadcasted_iota(jnp.int32, (NG, N), 0)
    m = (gi == bt).astype(jnp.float32)
    cnt_ref[...] = jnp.sum(m, axis=1, keepdims=True)


def _tc_prep(dps, dpd, batch2d):
    return pl.pallas_call(
        _tc_prep_body,
        out_shape=[jax.ShapeDtypeStruct((1, N), jnp.float32),
                   jax.ShapeDtypeStruct((1, N), jnp.float32),
                   jax.ShapeDtypeStruct((1, N), jnp.float32),
                   jax.ShapeDtypeStruct((1, N), jnp.int32),
                   jax.ShapeDtypeStruct((NG, 1), jnp.float32)],
    )(dps, dpd, batch2d)


def _sc_edges_body(src_hbm, dst_hbm, dinv_hbm, w_hbm, bN_hbm, zeros_hbm,
                   upart_hbm, mpart_hbm,
                   sb0, sb1, sb2, sb3, sb4, db0, db1, db2, db3, db4,
                   ci0, ci1, ci2, ci3, ci4, cv0, cv1, cv2, cv3, cv4,
                   dinv_v, w_v, bN_v, u_v, cshared,
                   sm0, sm1, sm2, sm3, sm4, cm0, cm1, cm2, cm3, cm4):
    sbufs = (sb0, sb1, sb2, sb3, sb4)
    dbufs = (db0, db1, db2, db3, db4)
    cidx = (ci0, ci1, ci2, ci3, ci4)
    cval = (cv0, cv1, cv2, cv3, cv4)
    sems = (sm0, sm1, sm2, sm3, sm4)
    csems = (cm0, cm1, cm2, cm3, cm4)
    cid = lax.axis_index("c")
    sid = lax.axis_index("s")
    wid = sid * NC + cid
    base = wid * EW
    psem = csems[0]
    pltpu.async_copy(dinv_hbm, dinv_v, psem)
    pltpu.async_copy(w_hbm, w_v, psem)
    pltpu.async_copy(bN_hbm, bN_v, psem)
    pltpu.async_copy(zeros_hbm, u_v, psem)
    for r in range(CSL // N):
        pltpu.async_copy(zeros_hbm, cshared.at[pl.ds(sid * CSL + r * N, N)],
                         psem)
    for b in range(RING):
        off = base + b * CCH
        pltpu.async_copy(src_hbm.at[pl.ds(off, CCH)], sbufs[b], sems[b])
        pltpu.async_copy(dst_hbm.at[pl.ds(off, CCH)], dbufs[b], sems[b])
    pltpu.make_async_copy(dinv_hbm, dinv_v, psem).wait()
    pltpu.make_async_copy(w_hbm, w_v, psem).wait()
    pltpu.make_async_copy(bN_hbm, bN_v, psem).wait()
    pltpu.make_async_copy(zeros_hbm, u_v, psem).wait()
    for r in range(CSL // N):
        pltpu.make_async_copy(zeros_hbm,
                              cshared.at[pl.ds(sid * CSL + r * N, N)],
                              psem).wait()
    plsc.subcore_barrier()

    def round_(k, _):
        for b in range(RING):
            off = base + (k * RING + b) * CCH
            pltpu.make_async_copy(src_hbm.at[pl.ds(off, CCH)], sbufs[b],
                                  sems[b]).wait()
            pltpu.make_async_copy(dst_hbm.at[pl.ds(off, CCH)], dbufs[b],
                                  sems[b]).wait()

            @pl.when(k > 0)
            def _():
                pltpu.make_async_copy(cval[b], cshared.at[cidx[b]],
                                      csems[b]).wait()

            for j in range(CVPC):
                s16 = sbufs[b][pl.ds(j * L, L)]
                d16 = dbufs[b][pl.ds(j * L, L)]
                ws = plsc.load_gather(w_v, [s16])
                dd = plsc.load_gather(dinv_v, [d16])
                bd = plsc.load_gather(bN_v, [d16])
                plsc.addupdate_scatter(u_v, [d16], ws)
                cidx[b][pl.ds(j * L, L)] = bd + s16
                cval[b][pl.ds(j * L, L)] = dd
            pltpu.async_copy(cval[b], cshared.at[cidx[b]], csems[b],
                             add=True)

            @pl.when(k < CNK - 1)
            def _():
                noff = base + ((k + 1) * RING + b) * CCH
                pltpu.async_copy(src_hbm.at[pl.ds(noff, CCH)], sbufs[b],
                                 sems[b])
                pltpu.async_copy(dst_hbm.at[pl.ds(noff, CCH)], dbufs[b],
                                 sems[b])
        return 0

    lax.fori_loop(0, CNK, round_, 0)
    for b in range(RING):
        pltpu.make_async_copy(cval[b], cshared.at[cidx[b]], csems[b]).wait()
    pltpu.sync_copy(u_v, upart_hbm.at[wid, pl.ds(0, N)])
    plsc.subcore_barrier()
    gpt = NG // NS
    for g in range(gpt):
        pltpu.async_copy(cshared.at[pl.ds((sid * gpt + g) * N, N)],
                         mpart_hbm.at[cid, sid * gpt + g, pl.ds(0, N)],
                         csems[1])
    for g in range(gpt):
        pltpu.make_async_copy(cshared.at[pl.ds((sid * gpt + g) * N, N)],
                              mpart_hbm.at[cid, sid * gpt + g, pl.ds(0, N)],
                              csems[1]).wait()


def _sc_edges(src, dst, dinv, w, bN, zeros_n):
    mesh = plsc.VectorSubcoreMesh(core_axis_name="c", subcore_axis_name="s")
    f = pl.kernel(
        _sc_edges_body,
        out_type=[jax.ShapeDtypeStruct((NW, NP), jnp.float32),
                  jax.ShapeDtypeStruct((NC, NG, NP), jnp.float32)],
        mesh=mesh,
        compiler_params=_SC_PARAMS,
        scratch_types=(
            [pltpu.VMEM((CCH,), jnp.int32) for _ in range(2 * RING)]
            + [pltpu.VMEM((CCH,), jnp.int32) for _ in range(RING)]
            + [pltpu.VMEM((CCH,), jnp.float32) for _ in range(RING)]
            + [pltpu.VMEM((N,), jnp.float32),
               pltpu.VMEM((N,), jnp.float32),
               pltpu.VMEM((N,), jnp.int32),
               pltpu.VMEM((N,), jnp.float32),
               pltpu.VMEM_SHARED((NG * N,), jnp.float32)]
            + [pltpu.SemaphoreType.DMA for _ in range(2 * RING)]
        ),
    )
    return f(src, dst, dinv, w, bN, zeros_n)


def _tc_final_body(upart_ref, mpart_ref, dinv_ref, dego_ref, batch_ref,
                   g1_ref, g0_ref, W2_ref, b2_ref, linW_ref, linb_ref,
                   cnt_ref, out_ref, q_acc):
    i = pl.program_id(0)

    @pl.when(i == 0)
    def _():
        q_acc[...] = jnp.zeros_like(q_acc)

    col = lax.broadcasted_iota(jnp.int32, (1, BN), 1) + i * BN
    valid = col < N
    dinv = dinv_ref[...]
    dinv2 = dinv * dinv
    u = jnp.sum(upart_ref[...], axis=0, keepdims=True)
    t = jnp.where(valid, dinv * u + dinv2 * dego_ref[...], 0.0)
    hT = jnp.maximum(g1_ref[...] * t + g0_ref[...], 0.0)
    gi = lax.broadcasted_iota(jnp.int32, (NG, BN), 0)
    m = (gi == batch_ref[...]).astype(jnp.float32)
    c_tot = jnp.where(valid,
                      (mpart_ref[0] + mpart_ref[1]) * dinv + m * dinv2, 0.0)
    q_acc[...] += lax.dot_general(
        c_tot, hT, (((1,), (1,)), ((), ())),
        preferred_element_type=jnp.float32)

    @pl.when(i == NSTEP - 1)
    def _():
        cnt = cnt_ref[...]
        maxcnt = jnp.maximum(cnt, 1.0)
        pooled = (jnp.dot(q_acc[...], W2_ref[...],
                          preferred_element_type=jnp.float32)
                  + cnt * b2_ref[...]) / maxcnt
        out_ref[...] = (jnp.dot(pooled, linW_ref[...],
                                preferred_element_type=jnp.float32)
                        + linb_ref[...])


def _tc_final(upart, mpart, dinv, dego, batch2d, g1c, g0c, W2, b2r, lin_W,
              lin_br, cnt):
    return pl.pallas_call(
        _tc_final_body,
        grid=(NSTEP,),
        in_specs=[
            pl.BlockSpec((NW, BN), lambda i: (0, i)),
            pl.BlockSpec((NC, NG, BN), lambda i: (0, 0, i)),
            pl.BlockSpec((1, BN), lambda i: (0, i)),
            pl.BlockSpec((1, BN), lambda i: (0, i)),
            pl.BlockSpec((1, BN), lambda i: (0, i)),
            pl.BlockSpec((HID, 1), lambda i: (0, 0)),
            pl.BlockSpec((HID, 1), lambda i: (0, 0)),
            pl.BlockSpec((HID, HID), lambda i: (0, 0)),
            pl.BlockSpec((1, HID), lambda i: (0, 0)),
            pl.BlockSpec((HID, OUT_DIM), lambda i: (0, 0)),
            pl.BlockSpec((1, OUT_DIM), lambda i: (0, 0)),
            pl.BlockSpec((NG, 1), lambda i: (0, 0)),
        ],
        out_specs=pl.BlockSpec((NG, OUT_DIM), lambda i: (0, 0)),
        out_shape=jax.ShapeDtypeStruct((NG, OUT_DIM), jnp.float32),
        scratch_shapes=[pltpu.VMEM((NG, HID), jnp.float32)],
    )(upart, mpart, dinv, dego, batch2d, g1c, g0c, W2, b2r, lin_W, lin_br,
      cnt)


def kernel(x, edge_index, batch, W1, b1, bn1_gamma, bn1_beta, bn1_mean,
           bn1_var, W2, b2, lin_W, lin_b):
    src = edge_index[0]
    dst = edge_index[1]
    zeros_n = jnp.zeros((N,), jnp.float32)

    dps, dpd = _sc_degrees(src, dst, zeros_n)
    dinv, dego, w, bN, cnt = _tc_prep(dps, dpd, batch.reshape(1, N))
    upart, mpart = _sc_edges(src, dst, dinv.reshape(N), w.reshape(N),
                             bN.reshape(N), zeros_n)

    bscale = bn1_gamma * lax.rsqrt(bn1_var + EPS)
    g1c = (W1[0] * bscale).reshape(HID, 1)
    g0c = ((b1 - bn1_mean) * bscale + bn1_beta).reshape(HID, 1)

    pad = NP - N
    dinv_p = jnp.pad(dinv, ((0, 0), (0, pad)))
    dego_p = jnp.pad(dego, ((0, 0), (0, pad)))
    batch_p = jnp.pad(batch.reshape(1, N), ((0, 0), (0, pad)),
                      constant_values=-1)

    return _tc_final(upart, mpart, dinv_p, dego_p, batch_p, g1c, g0c,
                     W2, b2.reshape(1, HID), lin_W, lin_b.reshape(1, OUT_DIM),
                     cnt)


# speedup vs baseline: 153.3996x; 1.5615x over previous
import jax
import jax.numpy as jnp
from jax import lax
from jax.experimental import pallas as pl
from jax.experimental.pallas import tpu as pltpu
from jax.experimental.pallas import tpu_sc as plsc

N = 10000
E = 320000
HID = 256
NG = 128
OUT_DIM = 128
EPS = 1e-5

NC = 2
NS = 16
NW = NC * NS
L = 16
RING = 5

EW = E // NW
ACH = 400
AVPC = ACH // L
NK = EW // (ACH * RING)
CCH = 80
SPC = ACH // CCH
CVPC = CCH // L
CSL = NG * N // NS
GPT = NG // NS

BN = 2048
NP = 10240
NSTEP = NP // BN
SLC = NP // NS

_SC_PARAMS = pltpu.CompilerParams(needs_layout_passes=False,
                                  use_tc_tiling_on_sc=False)
_MESH = dict(core_axis_name="c", subcore_axis_name="s")


def _sc_degrees_body(edge_hbm, dps_hbm, dpd_hbm,
                     sb0, sb1, sb2, sb3, sb4, db0, db1, db2, db3, db4,
                     dso_v, dsi_v, sm0, sm1, sm2, sm3, sm4):
    sbufs = (sb0, sb1, sb2, sb3, sb4)
    dbufs = (db0, db1, db2, db3, db4)
    sems = (sm0, sm1, sm2, sm3, sm4)
    cid = lax.axis_index("c")
    sid = lax.axis_index("s")
    wid = sid * NC + cid
    base = wid * EW
    for b in range(RING):
        off = base + b * ACH
        pltpu.async_copy(edge_hbm.at[pl.ds(off, ACH)], sbufs[b], sems[b])
        pltpu.async_copy(edge_hbm.at[pl.ds(E + off, ACH)], dbufs[b], sems[b])
    z16 = jnp.zeros((L,), jnp.float32)

    def zero_(i, _):
        dso_v[pl.ds(i * L, L)] = z16
        dsi_v[pl.ds(i * L, L)] = z16
        return 0

    lax.fori_loop(0, N // L, zero_, 0)
    ones = jnp.full((L,), 1.0, jnp.float32)

    def round_(k, _):
        for b in range(RING):
            off = base + (k * RING + b) * ACH
            pltpu.make_async_copy(edge_hbm.at[pl.ds(off, ACH)], sbufs[b],
                                  sems[b]).wait()
            pltpu.make_async_copy(edge_hbm.at[pl.ds(E + off, ACH)], dbufs[b],
                                  sems[b]).wait()
            for j in range(AVPC):
                s16 = sbufs[b][pl.ds(j * L, L)]
                d16 = dbufs[b][pl.ds(j * L, L)]
                plsc.addupdate_scatter(dso_v, [s16], ones)
                plsc.addupdate_scatter(dsi_v, [d16], ones)

            @pl.when(k < NK - 1)
            def _():
                noff = base + ((k + 1) * RING + b) * ACH
                pltpu.async_copy(edge_hbm.at[pl.ds(noff, ACH)], sbufs[b],
                                 sems[b])
                pltpu.async_copy(edge_hbm.at[pl.ds(E + noff, ACH)], dbufs[b],
                                 sems[b])
        return 0

    lax.fori_loop(0, NK, round_, 0)
    pltpu.sync_copy(dso_v, dps_hbm.at[pl.ds(wid * N, N)])
    pltpu.sync_copy(dsi_v, dpd_hbm.at[pl.ds(wid * N, N)])


def _sc_degrees(edge1d):
    f = pl.kernel(
        _sc_degrees_body,
        out_type=[jax.ShapeDtypeStruct((NW * N,), jnp.float32),
                  jax.ShapeDtypeStruct((NW * N,), jnp.float32)],
        mesh=plsc.VectorSubcoreMesh(**_MESH),
        compiler_params=_SC_PARAMS,
        scratch_types=(
            [pltpu.VMEM((ACH,), jnp.int32) for _ in range(2 * RING)]
            + [pltpu.VMEM((N,), jnp.float32) for _ in range(2)]
            + [pltpu.SemaphoreType.DMA for _ in range(RING)]
        ),
    )
    return f(edge1d)


def _tc_prep_body(dps_ref, dpd_ref, batch_ref, dinv_ref, dego_ref, w_ref,
                  bN_ref):
    dso = dps_ref[pl.ds(0, N)]
    dsi = dpd_ref[pl.ds(0, N)]
    for r in range(1, NW):
        dso = dso + dps_ref[pl.ds(r * N, N)]
        dsi = dsi + dpd_ref[pl.ds(r * N, N)]
    dinv = lax.rsqrt(dsi + 1.0)
    dego_ref[...] = dso
    dinv_ref[...] = dinv
    w_ref[...] = dinv * dso
    bN_ref[...] = batch_ref[...] * N


def _tc_prep(dps, dpd, batch):
    return pl.pallas_call(
        _tc_prep_body,
        out_shape=[jax.ShapeDtypeStruct((N,), jnp.float32),
                   jax.ShapeDtypeStruct((N,), jnp.float32),
                   jax.ShapeDtypeStruct((N,), jnp.float32),
                   jax.ShapeDtypeStruct((N,), jnp.int32)],
    )(dps, dpd, batch)


def _sc_edges_body(edge_hbm, dinv_hbm, w_hbm, bN_hbm,
                   u2_hbm, mpart_hbm,
                   sb0, sb1, sb2, sb3, sb4, db0, db1, db2, db3, db4,
                   ci0, ci1, ci2, ci3, ci4, cv0, cv1, cv2, cv3, cv4,
                   dinv_v, w_v, bN_v, u_v, cshared,
                   sm0, sm1, sm2, sm3, sm4, cm0, cm1, cm2, cm3, cm4, pm):
    sbufs = (sb0, sb1, sb2, sb3, sb4)
    dbufs = (db0, db1, db2, db3, db4)
    cidx = (ci0, ci1, ci2, ci3, ci4)
    cval = (cv0, cv1, cv2, cv3, cv4)
    sems = (sm0, sm1, sm2, sm3, sm4)
    csems = (cm0, cm1, cm2, cm3, cm4)
    cid = lax.axis_index("c")
    sid = lax.axis_index("s")
    wid = sid * NC + cid
    base = wid * EW
    pltpu.async_copy(dinv_hbm, dinv_v, pm)
    pltpu.async_copy(w_hbm, w_v, pm)
    pltpu.async_copy(bN_hbm, bN_v, pm)
    for b in range(RING):
        off = base + b * ACH
        pltpu.async_copy(edge_hbm.at[pl.ds(off, ACH)], sbufs[b], sems[b])
        pltpu.async_copy(edge_hbm.at[pl.ds(E + off, ACH)], dbufs[b], sems[b])
    z16 = jnp.zeros((L,), jnp.float32)

    def zero_(i, _):
        u_v[pl.ds(i * L, L)] = z16
        return 0

    lax.fori_loop(0, NP // L, zero_, 0)
    for r in range(CSL // N):
        pltpu.async_copy(u_v.at[pl.ds(0, N)],
                         cshared.at[pl.ds(sid * CSL + r * N, N)], pm)
    pltpu.make_async_copy(dinv_hbm, dinv_v, pm).wait()
    pltpu.make_async_copy(w_hbm, w_v, pm).wait()
    pltpu.make_async_copy(bN_hbm, bN_v, pm).wait()
    for r in range(CSL // N):
        pltpu.make_async_copy(u_v.at[pl.ds(0, N)],
                              cshared.at[pl.ds(sid * CSL + r * N, N)],
                              pm).wait()
    plsc.subcore_barrier()

    def round_(k, _):
        for b in range(RING):
            off = base + (k * RING + b) * ACH
            pltpu.make_async_copy(edge_hbm.at[pl.ds(off, ACH)], sbufs[b],
                                  sems[b]).wait()
            pltpu.make_async_copy(edge_hbm.at[pl.ds(E + off, ACH)], dbufs[b],
                                  sems[b]).wait()
            for sc in range(SPC):
                if b == 0:
                    @pl.when(k > 0)
                    def _(sc=sc):
                        pltpu.make_async_copy(cval[sc], cshared.at[cidx[sc]],
                                              csems[sc]).wait()
                else:
                    pltpu.make_async_copy(cval[sc], cshared.at[cidx[sc]],
                                          csems[sc]).wait()
                for j in range(CVPC):
                    o = sc * CCH + j * L
                    s16 = sbufs[b][pl.ds(o, L)]
                    d16 = dbufs[b][pl.ds(o, L)]
                    ws = plsc.load_gather(w_v, [s16])
                    dd = plsc.load_gather(dinv_v, [d16])
                    bd = plsc.load_gather(bN_v, [d16])
                    plsc.addupdate_scatter(u_v, [d16], ws)
                    cidx[sc][pl.ds(j * L, L)] = bd + s16
                    cval[sc][pl.ds(j * L, L)] = dd
                pltpu.async_copy(cval[sc], cshared.at[cidx[sc]], csems[sc],
                                 add=True)

            @pl.when(k < NK - 1)
            def _():
                noff = base + ((k + 1) * RING + b) * ACH
                pltpu.async_copy(edge_hbm.at[pl.ds(noff, ACH)], sbufs[b],
                                 sems[b])
                pltpu.async_copy(edge_hbm.at[pl.ds(E + noff, ACH)], dbufs[b],
                                 sems[b])
        return 0

    lax.fori_loop(0, NK, round_, 0)
    for sc in range(SPC):
        pltpu.make_async_copy(cval[sc], cshared.at[cidx[sc]],
                              csems[sc]).wait()
    plsc.subcore_barrier()
    for g in range(GPT):
        pltpu.async_copy(cshared.at[pl.ds((sid * GPT + g) * N, N)],
                         mpart_hbm.at[cid, sid * GPT + g, pl.ds(0, N)],
                         csems[1])
    for g in range(GPT):
        pltpu.make_async_copy(cshared.at[pl.ds((sid * GPT + g) * N, N)],
                              mpart_hbm.at[cid, sid * GPT + g, pl.ds(0, N)],
                              csems[1]).wait()
    plsc.subcore_barrier()
    pltpu.sync_copy(u_v, cshared.at[pl.ds(sid * NP, NP)])
    plsc.subcore_barrier()
    for slot in range(NS):
        pltpu.async_copy(cshared.at[pl.ds(slot * NP + sid * SLC, SLC)],
                         u_v.at[pl.ds(slot * SLC, SLC)], pm)
    for slot in range(NS):
        pltpu.make_async_copy(cshared.at[pl.ds(slot * NP + sid * SLC, SLC)],
                              u_v.at[pl.ds(slot * SLC, SLC)], pm).wait()
    for j in range(SLC // L):
        acc = u_v[pl.ds(j * L, L)]
        for slot in range(1, NS):
            acc = acc + u_v[pl.ds(slot * SLC + j * L, L)]
        u_v[pl.ds(j * L, L)] = acc
    pltpu.sync_copy(u_v.at[pl.ds(0, SLC)],
                    u2_hbm.at[cid, pl.ds(sid * SLC, SLC)])


def _sc_edges(edge1d, dinv, w, bN):
    f = pl.kernel(
        _sc_edges_body,
        out_type=[jax.ShapeDtypeStruct((NC, NP), jnp.float32),
                  jax.ShapeDtypeStruct((NC, NG, NP), jnp.float32)],
        mesh=plsc.VectorSubcoreMesh(**_MESH),
        compiler_params=_SC_PARAMS,
        scratch_types=(
            [pltpu.VMEM((ACH,), jnp.int32) for _ in range(2 * RING)]
            + [pltpu.VMEM((CCH,), jnp.int32) for _ in range(RING)]
            + [pltpu.VMEM((CCH,), jnp.float32) for _ in range(RING)]
            + [pltpu.VMEM((N,), jnp.float32),
               pltpu.VMEM((N,), jnp.float32),
               pltpu.VMEM((N,), jnp.int32),
               pltpu.VMEM((NP,), jnp.float32),
               pltpu.VMEM_SHARED((NG * N,), jnp.float32)]
            + [pltpu.SemaphoreType.DMA for _ in range(2 * RING + 1)]
        ),
    )
    return f(edge1d, dinv, w, bN)


def _tc_final_body(u2_ref, mpart_ref, dinv_ref, dego_ref, batch_ref,
                   g1_ref, g0_ref, W2_ref, b2_ref, linW_ref, linb_ref,
                   out_ref, q_acc, cnt_acc):
    i = pl.program_id(0)

    @pl.when(i == 0)
    def _():
        q_acc[...] = jnp.zeros_like(q_acc)
        cnt_acc[...] = jnp.zeros_like(cnt_acc)

    col = lax.broadcasted_iota(jnp.int32, (1, BN), 1) + i * BN
    valid = col < N
    dinv = dinv_ref[...]
    dinv2 = dinv * dinv
    u = jnp.sum(u2_ref[...], axis=0, keepdims=True)
    t = jnp.where(valid, dinv * u + dinv2 * dego_ref[...], 0.0)
    hT = jnp.maximum(g1_ref[...] * t + g0_ref[...], 0.0)
    gi = lax.broadcasted_iota(jnp.int32, (NG, BN), 0)
    m = (gi == batch_ref[...]).astype(jnp.float32)
    cnt_acc[...] += jnp.sum(m, axis=1, keepdims=True)
    c_tot = jnp.where(valid,
                      (mpart_ref[0] + mpart_ref[1]) * dinv + m * dinv2, 0.0)
    q_acc[...] += lax.dot_general(
        c_tot, hT, (((1,), (1,)), ((), ())),
        preferred_element_type=jnp.float32)

    @pl.when(i == NSTEP - 1)
    def _():
        cnt = cnt_acc[...]
        maxcnt = jnp.maximum(cnt, 1.0)
        pooled = (jnp.dot(q_acc[...], W2_ref[...],
                          preferred_element_type=jnp.float32)
                  + cnt * b2_ref[...]) / maxcnt
        out_ref[...] = (jnp.dot(pooled, linW_ref[...],
                                preferred_element_type=jnp.float32)
                        + linb_ref[...])


def _tc_final(u2, mpart, dinv, dego, batch2d, g1c, g0c, W2, b2r, lin_W,
              lin_br):
    return pl.pallas_call(
        _tc_final_body,
        grid=(NSTEP,),
        in_specs=[
            pl.BlockSpec((NC, BN), lambda i: (0, i)),
            pl.BlockSpec((NC, NG, BN), lambda i: (0, 0, i)),
            pl.BlockSpec((1, BN), lambda i: (0, i)),
            pl.BlockSpec((1, BN), lambda i: (0, i)),
            pl.BlockSpec((1, BN), lambda i: (0, i)),
            pl.BlockSpec((HID, 1), lambda i: (0, 0)),
            pl.BlockSpec((HID, 1), lambda i: (0, 0)),
            pl.BlockSpec((HID, HID), lambda i: (0, 0)),
            pl.BlockSpec((1, HID), lambda i: (0, 0)),
            pl.BlockSpec((HID, OUT_DIM), lambda i: (0, 0)),
            pl.BlockSpec((1, OUT_DIM), lambda i: (0, 0)),
        ],
        out_specs=pl.BlockSpec((NG, OUT_DIM), lambda i: (0, 0)),
        out_shape=jax.ShapeDtypeStruct((NG, OUT_DIM), jnp.float32),
        scratch_shapes=[pltpu.VMEM((NG, HID), jnp.float32),
                        pltpu.VMEM((NG, 1), jnp.float32)],
    )(u2, mpart, dinv, dego, batch2d, g1c, g0c, W2, b2r, lin_W, lin_br)


def kernel(x, edge_index, batch, W1, b1, bn1_gamma, bn1_beta, bn1_mean,
           bn1_var, W2, b2, lin_W, lin_b):
    edge1d = edge_index.reshape(2 * E)

    dps, dpd = _sc_degrees(edge1d)
    dinv, dego, w, bN = _tc_prep(dps, dpd, batch)
    u2, mpart = _sc_edges(edge1d, dinv, w, bN)

    bscale = bn1_gamma * lax.rsqrt(bn1_var + EPS)
    g1c = (W1[0] * bscale).reshape(HID, 1)
    g0c = ((b1 - bn1_mean) * bscale + bn1_beta).reshape(HID, 1)

    pad = NP - N
    dinv_p = jnp.pad(dinv, (0, pad)).reshape(1, NP)
    dego_p = jnp.pad(dego, (0, pad)).reshape(1, NP)
    batch_p = jnp.pad(batch, (0, pad), constant_values=-1).reshape(1, NP)

    return _tc_final(u2, mpart, dinv_p, dego_p, batch_p, g1c, g0c,
                     W2, b2.reshape(1, HID), lin_W, lin_b.reshape(1, OUT_DIM))


# speedup vs baseline: 156.4534x; 1.0199x over previous
import jax
import jax.numpy as jnp
from jax import lax
from jax.experimental import pallas as pl
from jax.experimental.pallas import tpu as pltpu
from jax.experimental.pallas import tpu_sc as plsc

N = 10000
E = 320000
HID = 256
NG = 128
OUT_DIM = 128
EPS = 1e-5

NC = 2
NS = 16
NW = NC * NS
L = 16
RING = 5

EW = E // NW
ACH = 400
AVPC = ACH // L
NK = EW // (ACH * RING)
CCH = 80
SPC = ACH // CCH
CVPC = CCH // L
CSL = NG * N // NS
GPT = NG // NS

BN = 2048
NP = 10240
NSTEP = NP // BN
SLC = NP // NS

_SC_PARAMS = pltpu.CompilerParams(needs_layout_passes=False,
                                  use_tc_tiling_on_sc=False)
_MESH = dict(core_axis_name="c", subcore_axis_name="s")


def _sc_degrees_body(edge_hbm, dps_hbm, dpd_hbm,
                     sb0, sb1, sb2, sb3, sb4, db0, db1, db2, db3, db4,
                     dso_v, dsi_v, sm0, sm1, sm2, sm3, sm4):
    sbufs = (sb0, sb1, sb2, sb3, sb4)
    dbufs = (db0, db1, db2, db3, db4)
    sems = (sm0, sm1, sm2, sm3, sm4)
    cid = lax.axis_index("c")
    sid = lax.axis_index("s")
    wid = sid * NC + cid
    base = wid * EW
    for b in range(RING):
        off = base + b * ACH
        pltpu.async_copy(edge_hbm.at[pl.ds(off, ACH)], sbufs[b], sems[b])
        pltpu.async_copy(edge_hbm.at[pl.ds(E + off, ACH)], dbufs[b], sems[b])
    z16 = jnp.zeros((L,), jnp.float32)

    def zero_(i, _):
        dso_v[pl.ds(i * L, L)] = z16
        dsi_v[pl.ds(i * L, L)] = z16
        return 0

    lax.fori_loop(0, N // L, zero_, 0)
    ones = jnp.full((L,), 1.0, jnp.float32)

    def round_(k, _):
        for b in range(RING):
            off = base + (k * RING + b) * ACH
            pltpu.make_async_copy(edge_hbm.at[pl.ds(off, ACH)], sbufs[b],
                                  sems[b]).wait()
            pltpu.make_async_copy(edge_hbm.at[pl.ds(E + off, ACH)], dbufs[b],
                                  sems[b]).wait()
            def vec_(j, _, b=b):
                s16 = sbufs[b][pl.ds(j * L, L)]
                d16 = dbufs[b][pl.ds(j * L, L)]
                plsc.addupdate_scatter(dso_v, [s16], ones)
                plsc.addupdate_scatter(dsi_v, [d16], ones)
                return 0

            lax.fori_loop(0, AVPC, vec_, 0)

            @pl.when(k < NK - 1)
            def _():
                noff = base + ((k + 1) * RING + b) * ACH
                pltpu.async_copy(edge_hbm.at[pl.ds(noff, ACH)], sbufs[b],
                                 sems[b])
                pltpu.async_copy(edge_hbm.at[pl.ds(E + noff, ACH)], dbufs[b],
                                 sems[b])
        return 0

    lax.fori_loop(0, NK, round_, 0)
    pltpu.sync_copy(dso_v, dps_hbm.at[pl.ds(wid * N, N)])
    pltpu.sync_copy(dsi_v, dpd_hbm.at[pl.ds(wid * N, N)])


def _sc_degrees(edge1d):
    f = pl.kernel(
        _sc_degrees_body,
        out_type=[jax.ShapeDtypeStruct((NW * N,), jnp.float32),
                  jax.ShapeDtypeStruct((NW * N,), jnp.float32)],
        mesh=plsc.VectorSubcoreMesh(**_MESH),
        compiler_params=_SC_PARAMS,
        scratch_types=(
            [pltpu.VMEM((ACH,), jnp.int32) for _ in range(2 * RING)]
            + [pltpu.VMEM((N,), jnp.float32) for _ in range(2)]
            + [pltpu.SemaphoreType.DMA for _ in range(RING)]
        ),
    )
    return f(edge1d)


def _tc_prep_body(dps_ref, dpd_ref, batch_ref, dinv_ref, dego_ref, w_ref,
                  bN_ref):
    dso = dps_ref[pl.ds(0, N)]
    dsi = dpd_ref[pl.ds(0, N)]
    for r in range(1, NW):
        dso = dso + dps_ref[pl.ds(r * N, N)]
        dsi = dsi + dpd_ref[pl.ds(r * N, N)]
    dinv = lax.rsqrt(dsi + 1.0)
    dego_ref[...] = dso
    dinv_ref[...] = dinv
    w_ref[...] = dinv * dso
    bN_ref[...] = batch_ref[...] * N


def _tc_prep(dps, dpd, batch):
    return pl.pallas_call(
        _tc_prep_body,
        out_shape=[jax.ShapeDtypeStruct((N,), jnp.float32),
                   jax.ShapeDtypeStruct((N,), jnp.float32),
                   jax.ShapeDtypeStruct((N,), jnp.float32),
                   jax.ShapeDtypeStruct((N,), jnp.int32)],
    )(dps, dpd, batch)


def _sc_edges_body(edge_hbm, dinv_hbm, w_hbm, bN_hbm,
                   u2_hbm, mpart_hbm,
                   sb0, sb1, sb2, sb3, sb4, db0, db1, db2, db3, db4,
                   ci0, ci1, ci2, ci3, ci4, cv0, cv1, cv2, cv3, cv4,
                   dinv_v, w_v, bN_v, u_v, cshared,
                   sm0, sm1, sm2, sm3, sm4, cm0, cm1, cm2, cm3, cm4, pm):
    sbufs = (sb0, sb1, sb2, sb3, sb4)
    dbufs = (db0, db1, db2, db3, db4)
    cidx = (ci0, ci1, ci2, ci3, ci4)
    cval = (cv0, cv1, cv2, cv3, cv4)
    sems = (sm0, sm1, sm2, sm3, sm4)
    csems = (cm0, cm1, cm2, cm3, cm4)
    cid = lax.axis_index("c")
    sid = lax.axis_index("s")
    wid = sid * NC + cid
    base = wid * EW
    pltpu.async_copy(dinv_hbm, dinv_v, pm)
    pltpu.async_copy(w_hbm, w_v, pm)
    pltpu.async_copy(bN_hbm, bN_v, pm)
    for b in range(RING):
        off = base + b * ACH
        pltpu.async_copy(edge_hbm.at[pl.ds(off, ACH)], sbufs[b], sems[b])
        pltpu.async_copy(edge_hbm.at[pl.ds(E + off, ACH)], dbufs[b], sems[b])
    z16 = jnp.zeros((L,), jnp.float32)

    def zero_(i, _):
        u_v[pl.ds(i * L, L)] = z16
        return 0

    lax.fori_loop(0, NP // L, zero_, 0)
    for r in range(CSL // N):
        pltpu.async_copy(u_v.at[pl.ds(0, N)],
                         cshared.at[pl.ds(sid * CSL + r * N, N)], pm)
    pltpu.make_async_copy(dinv_hbm, dinv_v, pm).wait()
    pltpu.make_async_copy(w_hbm, w_v, pm).wait()
    pltpu.make_async_copy(bN_hbm, bN_v, pm).wait()
    for r in range(CSL // N):
        pltpu.make_async_copy(u_v.at[pl.ds(0, N)],
                              cshared.at[pl.ds(sid * CSL + r * N, N)],
                              pm).wait()
    plsc.subcore_barrier()

    def round_(k, _):
        for b in range(RING):
            off = base + (k * RING + b) * ACH
            pltpu.make_async_copy(edge_hbm.at[pl.ds(off, ACH)], sbufs[b],
                                  sems[b]).wait()
            pltpu.make_async_copy(edge_hbm.at[pl.ds(E + off, ACH)], dbufs[b],
                                  sems[b]).wait()
            for sc in range(SPC):
                if b == 0:
                    @pl.when(k > 0)
                    def _(sc=sc):
                        pltpu.make_async_copy(cval[sc], cshared.at[cidx[sc]],
                                              csems[sc]).wait()
                else:
                    pltpu.make_async_copy(cval[sc], cshared.at[cidx[sc]],
                                          csems[sc]).wait()
                def vec_(j, _, b=b, sc=sc):
                    o = sc * CCH + j * L
                    s16 = sbufs[b][pl.ds(o, L)]
                    d16 = dbufs[b][pl.ds(o, L)]
                    ws = plsc.load_gather(w_v, [s16])
                    dd = plsc.load_gather(dinv_v, [d16])
                    bd = plsc.load_gather(bN_v, [d16])
                    plsc.addupdate_scatter(u_v, [d16], ws)
                    cidx[sc][pl.ds(j * L, L)] = bd + s16
                    cval[sc][pl.ds(j * L, L)] = dd
                    return 0

                lax.fori_loop(0, CVPC, vec_, 0)
                pltpu.async_copy(cval[sc], cshared.at[cidx[sc]], csems[sc],
                                 add=True)

            @pl.when(k < NK - 1)
            def _():
                noff = base + ((k + 1) * RING + b) * ACH
                pltpu.async_copy(edge_hbm.at[pl.ds(noff, ACH)], sbufs[b],
                                 sems[b])
                pltpu.async_copy(edge_hbm.at[pl.ds(E + noff, ACH)], dbufs[b],
                                 sems[b])
        return 0

    lax.fori_loop(0, NK, round_, 0)
    for sc in range(SPC):
        pltpu.make_async_copy(cval[sc], cshared.at[cidx[sc]],
                              csems[sc]).wait()
    plsc.subcore_barrier()
    for g in range(GPT):
        pltpu.async_copy(cshared.at[pl.ds((sid * GPT + g) * N, N)],
                         mpart_hbm.at[cid, sid * GPT + g, pl.ds(0, N)],
                         csems[1])
    for g in range(GPT):
        pltpu.make_async_copy(cshared.at[pl.ds((sid * GPT + g) * N, N)],
                              mpart_hbm.at[cid, sid * GPT + g, pl.ds(0, N)],
                              csems[1]).wait()
    plsc.subcore_barrier()
    pltpu.sync_copy(u_v, cshared.at[pl.ds(sid * NP, NP)])
    plsc.subcore_barrier()
    for slot in range(NS):
        pltpu.async_copy(cshared.at[pl.ds(slot * NP + sid * SLC, SLC)],
                         u_v.at[pl.ds(slot * SLC, SLC)], pm)
    for slot in range(NS):
        pltpu.make_async_copy(cshared.at[pl.ds(slot * NP + sid * SLC, SLC)],
                              u_v.at[pl.ds(slot * SLC, SLC)], pm).wait()
    def red_(j, _):
        acc = u_v[pl.ds(j * L, L)]
        for slot in range(1, NS):
            acc = acc + u_v[pl.ds(slot * SLC + j * L, L)]
        u_v[pl.ds(j * L, L)] = acc
        return 0

    lax.fori_loop(0, SLC // L, red_, 0)
    pltpu.sync_copy(u_v.at[pl.ds(0, SLC)],
                    u2_hbm.at[cid, pl.ds(sid * SLC, SLC)])


def _sc_edges(edge1d, dinv, w, bN):
    f = pl.kernel(
        _sc_edges_body,
        out_type=[jax.ShapeDtypeStruct((NC, NP), jnp.float32),
                  jax.ShapeDtypeStruct((NC, NG, NP), jnp.float32)],
        mesh=plsc.VectorSubcoreMesh(**_MESH),
        compiler_params=_SC_PARAMS,
        scratch_types=(
            [pltpu.VMEM((ACH,), jnp.int32) for _ in range(2 * RING)]
            + [pltpu.VMEM((CCH,), jnp.int32) for _ in range(RING)]
            + [pltpu.VMEM((CCH,), jnp.float32) for _ in range(RING)]
            + [pltpu.VMEM((N,), jnp.float32),
               pltpu.VMEM((N,), jnp.float32),
               pltpu.VMEM((N,), jnp.int32),
               pltpu.VMEM((NP,), jnp.float32),
               pltpu.VMEM_SHARED((NG * N,), jnp.float32)]
            + [pltpu.SemaphoreType.DMA for _ in range(2 * RING + 1)]
        ),
    )
    return f(edge1d, dinv, w, bN)


def _tc_final_body(u2_ref, mpart_ref, dinv_ref, dego_ref, batch_ref,
                   g1_ref, g0_ref, W2_ref, b2_ref, linW_ref, linb_ref,
                   out_ref, q_acc, cnt_acc):
    i = pl.program_id(0)

    @pl.when(i == 0)
    def _():
        q_acc[...] = jnp.zeros_like(q_acc)
        cnt_acc[...] = jnp.zeros_like(cnt_acc)

    col = lax.broadcasted_iota(jnp.int32, (1, BN), 1) + i * BN
    valid = col < N
    dinv = dinv_ref[...]
    dinv2 = dinv * dinv
    u = jnp.sum(u2_ref[...], axis=0, keepdims=True)
    t = jnp.where(valid, dinv * u + dinv2 * dego_ref[...], 0.0)
    hT = jnp.maximum(g1_ref[...] * t + g0_ref[...], 0.0)
    gi = lax.broadcasted_iota(jnp.int32, (NG, BN), 0)
    m = (gi == batch_ref[...]).astype(jnp.float32)
    cnt_acc[...] += jnp.sum(m, axis=1, keepdims=True)
    c_tot = jnp.where(valid,
                      (mpart_ref[0] + mpart_ref[1]) * dinv + m * dinv2, 0.0)
    q_acc[...] += lax.dot_general(
        c_tot, hT, (((1,), (1,)), ((), ())),
        preferred_element_type=jnp.float32)

    @pl.when(i == NSTEP - 1)
    def _():
        cnt = cnt_acc[...]
        maxcnt = jnp.maximum(cnt, 1.0)
        pooled = (jnp.dot(q_acc[...], W2_ref[...],
                          preferred_element_type=jnp.float32)
                  + cnt * b2_ref[...]) / maxcnt
        out_ref[...] = (jnp.dot(pooled, linW_ref[...],
                                preferred_element_type=jnp.float32)
                        + linb_ref[...])


def _tc_final(u2, mpart, dinv, dego, batch2d, g1c, g0c, W2, b2r, lin_W,
              lin_br):
    return pl.pallas_call(
        _tc_final_body,
        grid=(NSTEP,),
        in_specs=[
            pl.BlockSpec((NC, BN), lambda i: (0, i)),
            pl.BlockSpec((NC, NG, BN), lambda i: (0, 0, i)),
            pl.BlockSpec((1, BN), lambda i: (0, i)),
            pl.BlockSpec((1, BN), lambda i: (0, i)),
            pl.BlockSpec((1, BN), lambda i: (0, i)),
            pl.BlockSpec((HID, 1), lambda i: (0, 0)),
            pl.BlockSpec((HID, 1), lambda i: (0, 0)),
            pl.BlockSpec((HID, HID), lambda i: (0, 0)),
            pl.BlockSpec((1, HID), lambda i: (0, 0)),
            pl.BlockSpec((HID, OUT_DIM), lambda i: (0, 0)),
            pl.BlockSpec((1, OUT_DIM), lambda i: (0, 0)),
        ],
        out_specs=pl.BlockSpec((NG, OUT_DIM), lambda i: (0, 0)),
        out_shape=jax.ShapeDtypeStruct((NG, OUT_DIM), jnp.float32),
        scratch_shapes=[pltpu.VMEM((NG, HID), jnp.float32),
                        pltpu.VMEM((NG, 1), jnp.float32)],
    )(u2, mpart, dinv, dego, batch2d, g1c, g0c, W2, b2r, lin_W, lin_br)


def kernel(x, edge_index, batch, W1, b1, bn1_gamma, bn1_beta, bn1_mean,
           bn1_var, W2, b2, lin_W, lin_b):
    edge1d = edge_index.reshape(2 * E)

    dps, dpd = _sc_degrees(edge1d)
    dinv, dego, w, bN = _tc_prep(dps, dpd, batch)
    u2, mpart = _sc_edges(edge1d, dinv, w, bN)

    bscale = bn1_gamma * lax.rsqrt(bn1_var + EPS)
    g1c = (W1[0] * bscale).reshape(HID, 1)
    g0c = ((b1 - bn1_mean) * bscale + bn1_beta).reshape(HID, 1)

    pad = NP - N
    dinv_p = jnp.pad(dinv, (0, pad)).reshape(1, NP)
    dego_p = jnp.pad(dego, (0, pad)).reshape(1, NP)
    batch_p = jnp.pad(batch, (0, pad), constant_values=-1).reshape(1, NP)

    return _tc_final(u2, mpart, dinv_p, dego_p, batch_p, g1c, g0c,
                     W2, b2.reshape(1, HID), lin_W, lin_b.reshape(1, OUT_DIM))


# speedup vs baseline: 156.7872x; 1.0021x over previous
import jax
import jax.numpy as jnp
from jax import lax
from jax.experimental import pallas as pl
from jax.experimental.pallas import tpu as pltpu
from jax.experimental.pallas import tpu_sc as plsc

N = 10000
E = 320000
HID = 256
NG = 128
OUT_DIM = 128
EPS = 1e-5

NC = 2
NS = 16
NW = NC * NS
L = 16
RING = 5

EW = E // NW
ACH = 400
AVPC = ACH // L
NK = EW // (ACH * RING)
CCH = 80
SPC = ACH // CCH
CVPC = CCH // L
CSL = NG * N // NS
GPT = NG // NS

BN = 2048
NP = 10240
NSTEP = NP // BN
SLC = NP // NS

_SC_PARAMS = pltpu.CompilerParams(needs_layout_passes=False,
                                  use_tc_tiling_on_sc=False)
_MESH = dict(core_axis_name="c", subcore_axis_name="s")


def _sc_degrees_body(edge_hbm, dps_hbm, dpd_hbm,
                     sb0, sb1, sb2, sb3, sb4, db0, db1, db2, db3, db4,
                     dso_v, dsi_v, sm0, sm1, sm2, sm3, sm4):
    sbufs = (sb0, sb1, sb2, sb3, sb4)
    dbufs = (db0, db1, db2, db3, db4)
    sems = (sm0, sm1, sm2, sm3, sm4)
    cid = lax.axis_index("c")
    sid = lax.axis_index("s")
    wid = sid * NC + cid
    base = wid * EW
    for b in range(RING):
        off = base + b * ACH
        pltpu.async_copy(edge_hbm.at[pl.ds(off, ACH)], sbufs[b], sems[b])
        pltpu.async_copy(edge_hbm.at[pl.ds(E + off, ACH)], dbufs[b], sems[b])
    z16 = jnp.zeros((L,), jnp.float32)

    def zero_(i, _):
        dso_v[pl.ds(i * L, L)] = z16
        dsi_v[pl.ds(i * L, L)] = z16
        return 0

    lax.fori_loop(0, N // L, zero_, 0)
    ones = jnp.full((L,), 1.0, jnp.float32)

    def round_(k, _):
        for b in range(RING):
            off = base + (k * RING + b) * ACH
            pltpu.make_async_copy(edge_hbm.at[pl.ds(off, ACH)], sbufs[b],
                                  sems[b]).wait()
            pltpu.make_async_copy(edge_hbm.at[pl.ds(E + off, ACH)], dbufs[b],
                                  sems[b]).wait()
            def vec_(j, _, b=b):
                s16 = sbufs[b][pl.ds(j * L, L)]
                d16 = dbufs[b][pl.ds(j * L, L)]
                plsc.addupdate_scatter(dso_v, [s16], ones)
                plsc.addupdate_scatter(dsi_v, [d16], ones)
                return 0

            lax.fori_loop(0, AVPC, vec_, 0)

            @pl.when(k < NK - 1)
            def _():
                noff = base + ((k + 1) * RING + b) * ACH
                pltpu.async_copy(edge_hbm.at[pl.ds(noff, ACH)], sbufs[b],
                                 sems[b])
                pltpu.async_copy(edge_hbm.at[pl.ds(E + noff, ACH)], dbufs[b],
                                 sems[b])
        return 0

    lax.fori_loop(0, NK, round_, 0)
    pltpu.sync_copy(dso_v, dps_hbm.at[pl.ds(wid * N, N)])
    pltpu.sync_copy(dsi_v, dpd_hbm.at[pl.ds(wid * N, N)])


def _sc_degrees(edge1d):
    f = pl.kernel(
        _sc_degrees_body,
        out_type=[jax.ShapeDtypeStruct((NW * N,), jnp.float32),
                  jax.ShapeDtypeStruct((NW * N,), jnp.float32)],
        mesh=plsc.VectorSubcoreMesh(**_MESH),
        compiler_params=_SC_PARAMS,
        scratch_types=(
            [pltpu.VMEM((ACH,), jnp.int32) for _ in range(2 * RING)]
            + [pltpu.VMEM((N,), jnp.float32) for _ in range(2)]
            + [pltpu.SemaphoreType.DMA for _ in range(RING)]
        ),
    )
    return f(edge1d)


def _tc_prep_body(dps_ref, dpd_ref, batch_ref, dinv_ref, dego_ref, w_ref,
                  bN_ref):
    dso = dps_ref[pl.ds(0, N)]
    dsi = dpd_ref[pl.ds(0, N)]
    for r in range(1, NW):
        dso = dso + dps_ref[pl.ds(r * N, N)]
        dsi = dsi + dpd_ref[pl.ds(r * N, N)]
    dinv = lax.rsqrt(dsi + 1.0)
    dego_ref[...] = dso
    dinv_ref[...] = dinv
    w_ref[...] = dinv * dso
    bN_ref[...] = batch_ref[...] * N


def _tc_prep(dps, dpd, batch):
    return pl.pallas_call(
        _tc_prep_body,
        out_shape=[jax.ShapeDtypeStruct((N,), jnp.float32),
                   jax.ShapeDtypeStruct((N,), jnp.float32),
                   jax.ShapeDtypeStruct((N,), jnp.float32),
                   jax.ShapeDtypeStruct((N,), jnp.int32)],
    )(dps, dpd, batch)


def _sc_edges_body(edge_hbm, dinv_hbm, w_hbm, bN_hbm,
                   u2_hbm, mpart_hbm,
                   sb0, sb1, sb2, sb3, sb4, db0, db1, db2, db3, db4,
                   ci0, ci1, ci2, ci3, ci4, cv0, cv1, cv2, cv3, cv4,
                   dinv_v, w_v, bN_v, u_v, cshared,
                   sm0, sm1, sm2, sm3, sm4, cm0, cm1, cm2, cm3, cm4, pm):
    sbufs = (sb0, sb1, sb2, sb3, sb4)
    dbufs = (db0, db1, db2, db3, db4)
    cidx = (ci0, ci1, ci2, ci3, ci4)
    cval = (cv0, cv1, cv2, cv3, cv4)
    sems = (sm0, sm1, sm2, sm3, sm4)
    csems = (cm0, cm1, cm2, cm3, cm4)
    cid = lax.axis_index("c")
    sid = lax.axis_index("s")
    wid = sid * NC + cid
    base = wid * EW
    pltpu.async_copy(dinv_hbm, dinv_v, pm)
    pltpu.async_copy(w_hbm, w_v, pm)
    pltpu.async_copy(bN_hbm, bN_v, pm)
    for b in range(RING):
        off = base + b * ACH
        pltpu.async_copy(edge_hbm.at[pl.ds(off, ACH)], sbufs[b], sems[b])
        pltpu.async_copy(edge_hbm.at[pl.ds(E + off, ACH)], dbufs[b], sems[b])
    z16 = jnp.zeros((L,), jnp.float32)

    def zero_(i, _):
        u_v[pl.ds(i * L, L)] = z16
        return 0

    lax.fori_loop(0, NP // L, zero_, 0)
    for r in range(CSL // N):
        pltpu.async_copy(u_v.at[pl.ds(0, N)],
                         cshared.at[pl.ds(sid * CSL + r * N, N)], pm)
    pltpu.make_async_copy(dinv_hbm, dinv_v, pm).wait()
    pltpu.make_async_copy(w_hbm, w_v, pm).wait()
    pltpu.make_async_copy(bN_hbm, bN_v, pm).wait()
    for r in range(CSL // N):
        pltpu.make_async_copy(u_v.at[pl.ds(0, N)],
                              cshared.at[pl.ds(sid * CSL + r * N, N)],
                              pm).wait()
    plsc.subcore_barrier()

    def round_(k, _):
        for b in range(RING):
            off = base + (k * RING + b) * ACH
            pltpu.make_async_copy(edge_hbm.at[pl.ds(off, ACH)], sbufs[b],
                                  sems[b]).wait()
            pltpu.make_async_copy(edge_hbm.at[pl.ds(E + off, ACH)], dbufs[b],
                                  sems[b]).wait()
            for sc in range(SPC):
                if b == 0:
                    @pl.when(k > 0)
                    def _(sc=sc):
                        pltpu.make_async_copy(cval[sc], cshared.at[cidx[sc]],
                                              csems[sc]).wait()
                else:
                    pltpu.make_async_copy(cval[sc], cshared.at[cidx[sc]],
                                          csems[sc]).wait()
                def vec_(j, _, b=b, sc=sc):
                    o = sc * CCH + j * L
                    s16 = sbufs[b][pl.ds(o, L)]
                    d16 = dbufs[b][pl.ds(o, L)]
                    ws = plsc.load_gather(w_v, [s16])
                    dd = plsc.load_gather(dinv_v, [d16])
                    bd = plsc.load_gather(bN_v, [d16])
                    plsc.addupdate_scatter(u_v, [d16], ws)
                    cidx[sc][pl.ds(j * L, L)] = bd + s16
                    cval[sc][pl.ds(j * L, L)] = dd
                    return 0

                lax.fori_loop(0, CVPC, vec_, 0)
                pltpu.async_copy(cval[sc], cshared.at[cidx[sc]], csems[sc],
                                 add=True)

            @pl.when(k < NK - 1)
            def _():
                noff = base + ((k + 1) * RING + b) * ACH
                pltpu.async_copy(edge_hbm.at[pl.ds(noff, ACH)], sbufs[b],
                                 sems[b])
                pltpu.async_copy(edge_hbm.at[pl.ds(E + noff, ACH)], dbufs[b],
                                 sems[b])
        return 0

    lax.fori_loop(0, NK, round_, 0)
    for sc in range(SPC):
        pltpu.make_async_copy(cval[sc], cshared.at[cidx[sc]],
                              csems[sc]).wait()
    plsc.subcore_barrier()
    for g in range(GPT):
        pltpu.async_copy(cshared.at[pl.ds((sid * GPT + g) * N, N)],
                         mpart_hbm.at[cid, sid * GPT + g, pl.ds(0, N)],
                         csems[1])
    for g in range(GPT):
        pltpu.make_async_copy(cshared.at[pl.ds((sid * GPT + g) * N, N)],
                              mpart_hbm.at[cid, sid * GPT + g, pl.ds(0, N)],
                              csems[1]).wait()
    plsc.subcore_barrier()
    pltpu.sync_copy(u_v, cshared.at[pl.ds(sid * NP, NP)])
    plsc.subcore_barrier()
    for slot in range(NS):
        pltpu.async_copy(cshared.at[pl.ds(slot * NP + sid * SLC, SLC)],
                         u_v.at[pl.ds(slot * SLC, SLC)], pm)
    for slot in range(NS):
        pltpu.make_async_copy(cshared.at[pl.ds(slot * NP + sid * SLC, SLC)],
                              u_v.at[pl.ds(slot * SLC, SLC)], pm).wait()
    def red_(j, _):
        acc = u_v[pl.ds(j * L, L)]
        for slot in range(1, NS):
            acc = acc + u_v[pl.ds(slot * SLC + j * L, L)]
        u_v[pl.ds(j * L, L)] = acc
        return 0

    lax.fori_loop(0, SLC // L, red_, 0)
    pltpu.sync_copy(u_v.at[pl.ds(0, SLC)],
                    u2_hbm.at[cid, pl.ds(sid * SLC, SLC)])


def _sc_edges(edge1d, dinv, w, bN):
    f = pl.kernel(
        _sc_edges_body,
        out_type=[jax.ShapeDtypeStruct((NC, NP), jnp.float32),
                  jax.ShapeDtypeStruct((NC, NG, NP), jnp.float32)],
        mesh=plsc.VectorSubcoreMesh(**_MESH),
        compiler_params=_SC_PARAMS,
        scratch_types=(
            [pltpu.VMEM((ACH,), jnp.int32) for _ in range(2 * RING)]
            + [pltpu.VMEM((CCH,), jnp.int32) for _ in range(RING)]
            + [pltpu.VMEM((CCH,), jnp.float32) for _ in range(RING)]
            + [pltpu.VMEM((N,), jnp.float32),
               pltpu.VMEM((N,), jnp.float32),
               pltpu.VMEM((N,), jnp.int32),
               pltpu.VMEM((NP,), jnp.float32),
               pltpu.VMEM_SHARED((NG * N,), jnp.float32)]
            + [pltpu.SemaphoreType.DMA for _ in range(2 * RING + 1)]
        ),
    )
    return f(edge1d, dinv, w, bN)


def _tc_final_body(u2_ref, mpart_ref, dinv_ref, dego_ref, batch_ref,
                   g1_ref, g0_ref, W2_ref, b2_ref, linW_ref, linb_ref,
                   out_ref, q_acc, cnt_acc):
    i = pl.program_id(0)

    @pl.when(i == 0)
    def _():
        q_acc[...] = jnp.zeros_like(q_acc)
        cnt_acc[...] = jnp.zeros_like(cnt_acc)

    col = lax.broadcasted_iota(jnp.int32, (1, BN), 1) + i * BN
    valid = col < N
    dinv = dinv_ref[...]
    dinv2 = dinv * dinv
    u = jnp.sum(u2_ref[...], axis=0, keepdims=True)
    t = jnp.where(valid, dinv * u + dinv2 * dego_ref[...], 0.0)
    hT = jnp.maximum(g1_ref[...] * t + g0_ref[...], 0.0)
    gi = lax.broadcasted_iota(jnp.int32, (NG, BN), 0)
    m = (gi == batch_ref[...]).astype(jnp.float32)
    cnt_acc[...] += jnp.sum(m, axis=1, keepdims=True)
    c_tot = jnp.where(valid,
                      (mpart_ref[0] + mpart_ref[1]) * dinv + m * dinv2, 0.0)
    q_acc[...] += lax.dot_general(
        c_tot.astype(jnp.bfloat16), hT.astype(jnp.bfloat16),
        (((1,), (1,)), ((), ())),
        preferred_element_type=jnp.float32)

    @pl.when(i == NSTEP - 1)
    def _():
        cnt = cnt_acc[...]
        maxcnt = jnp.maximum(cnt, 1.0)
        pooled = (jnp.dot(q_acc[...], W2_ref[...],
                          preferred_element_type=jnp.float32)
                  + cnt * b2_ref[...]) / maxcnt
        out_ref[...] = (jnp.dot(pooled, linW_ref[...],
                                preferred_element_type=jnp.float32)
                        + linb_ref[...])


def _tc_final(u2, mpart, dinv, dego, batch2d, g1c, g0c, W2, b2r, lin_W,
              lin_br):
    return pl.pallas_call(
        _tc_final_body,
        grid=(NSTEP,),
        in_specs=[
            pl.BlockSpec((NC, BN), lambda i: (0, i)),
            pl.BlockSpec((NC, NG, BN), lambda i: (0, 0, i)),
            pl.BlockSpec((1, BN), lambda i: (0, i)),
            pl.BlockSpec((1, BN), lambda i: (0, i)),
            pl.BlockSpec((1, BN), lambda i: (0, i)),
            pl.BlockSpec((HID, 1), lambda i: (0, 0)),
            pl.BlockSpec((HID, 1), lambda i: (0, 0)),
            pl.BlockSpec((HID, HID), lambda i: (0, 0)),
            pl.BlockSpec((1, HID), lambda i: (0, 0)),
            pl.BlockSpec((HID, OUT_DIM), lambda i: (0, 0)),
            pl.BlockSpec((1, OUT_DIM), lambda i: (0, 0)),
        ],
        out_specs=pl.BlockSpec((NG, OUT_DIM), lambda i: (0, 0)),
        out_shape=jax.ShapeDtypeStruct((NG, OUT_DIM), jnp.float32),
        scratch_shapes=[pltpu.VMEM((NG, HID), jnp.float32),
                        pltpu.VMEM((NG, 1), jnp.float32)],
    )(u2, mpart, dinv, dego, batch2d, g1c, g0c, W2, b2r, lin_W, lin_br)


def kernel(x, edge_index, batch, W1, b1, bn1_gamma, bn1_beta, bn1_mean,
           bn1_var, W2, b2, lin_W, lin_b):
    edge1d = edge_index.reshape(2 * E)

    dps, dpd = _sc_degrees(edge1d)
    dinv, dego, w, bN = _tc_prep(dps, dpd, batch)
    u2, mpart = _sc_edges(edge1d, dinv, w, bN)

    bscale = bn1_gamma * lax.rsqrt(bn1_var + EPS)
    g1c = (W1[0] * bscale).reshape(HID, 1)
    g0c = ((b1 - bn1_mean) * bscale + bn1_beta).reshape(HID, 1)

    pad = NP - N
    dinv_p = jnp.pad(dinv, (0, pad)).reshape(1, NP)
    dego_p = jnp.pad(dego, (0, pad)).reshape(1, NP)
    batch_p = jnp.pad(batch, (0, pad), constant_values=-1).reshape(1, NP)

    return _tc_final(u2, mpart, dinv_p, dego_p, batch_p, g1c, g0c,
                     W2, b2.reshape(1, HID), lin_W, lin_b.reshape(1, OUT_DIM))
